# baseline (speedup 1.0000x reference)
import functools

import jax
import jax.numpy as jnp
import numpy as np
from jax import lax
from jax.experimental import pallas as pl
from jax.experimental.pallas import tpu as pltpu
from jax.experimental.pallas import tpu_sc as plsc

N = 10000
E = 320000
D = 128
BN_EPS = 1e-3

NC = 2
NS = 16
NW = NC * NS
EPW = E // NW
CH = 80
NFULL = EPW // CH
NB = 3
WB = 640

BE = 2000
BNODE = 2000

_PERM = np.arange(D).reshape(D // 32, 2, 16).transpose(0, 2, 1).reshape(D)


def _he_body(ea_ref, w_ref, b_ref, out_ref):
    out_ref[...] = (
        jnp.dot(ea_ref[...], w_ref[...], preferred_element_type=jnp.float32)
        + b_ref[...]
    ).astype(jnp.bfloat16)


def _compute_he(edge_attr, w, b):
    return pl.pallas_call(
        _he_body,
        grid=(E // BE,),
        in_specs=[
            pl.BlockSpec((BE, 16), lambda i: (i, 0)),
            pl.BlockSpec((16, D), lambda i: (0, 0)),
            pl.BlockSpec((1, D), lambda i: (0, 0)),
        ],
        out_specs=pl.BlockSpec((BE, D), lambda i: (i, 0)),
        out_shape=jax.ShapeDtypeStruct((E, D), jnp.bfloat16),
    )(edge_attr, w, b)


def _sc_pool_body(idx_hbm, he_hbm, x_hbm, out_hbm,
                  idx_v, xr_v, he_v0, he_v1, he_v2, pooled_sh,
                  isem0, isem1, isem2, gsem0, gsem1, gsem2,
                  hsem0, hsem1, hsem2):
    cid = lax.axis_index("c")
    sid = lax.axis_index("s")
    wid = sid * NC + cid
    he_vs = (he_v0, he_v1, he_v2)
    isems = (isem0, isem1, isem2)
    gsems = (gsem0, gsem1, gsem2)
    hsems = (hsem0, hsem1, hsem2)

    def zrow(r, carry):
        for c in range(D // 16):
            xr_v[0, r, pl.ds(c * 16, 16)] = jnp.zeros((16,), jnp.float32)
        return carry
    lax.fori_loop(0, CH, zrow, 0)
    for k in range(WB // CH):
        r0 = sid * WB + k * CH

        @pl.when(r0 + CH <= N)
        def _():
            pltpu.sync_copy(xr_v.at[0], pooled_sh.at[pl.ds(r0, CH)])

    plsc.subcore_barrier()

    cbase = wid * NFULL

    def start_idx(c, b):
        pltpu.async_copy(idx_hbm.at[cbase + c], idx_v.at[b], isems[b])

    def wait_idx(b):
        pltpu.make_async_copy(idx_hbm.at[cbase], idx_v.at[b],
                              isems[b]).wait()

    def start_data(c, b):
        pltpu.async_copy(x_hbm.at[idx_v.at[b, 0]], xr_v.at[b], gsems[b])
        pltpu.async_copy(
            he_hbm.at[pl.ds((cbase + c) * (CH * D // 2), CH * D // 2)],
            he_vs[b], hsems[b])

    def process_chunk(c, b):
        b2 = (b + 2) % NB
        pltpu.make_async_copy(x_hbm.at[idx_v.at[b, 0]], xr_v.at[b],
                              gsems[b]).wait()
        pltpu.make_async_copy(he_hbm.at[pl.ds(0, CH * D // 2)], he_vs[b],
                              hsems[b]).wait()

        @pl.when(c + 2 < NFULL)
        def _():
            wait_idx(b2)
            start_data(c + 2, b2)

        def row(r, c2):
            rH = r * (D // 2)
            for k in range(D // 32):
                s = k * 32
                hw = he_vs[b][pl.ds(rH + k * 16, 16)]
                lo = lax.bitcast_convert_type(hw << 16, jnp.float32)
                hi = lax.bitcast_convert_type(hw & jnp.int32(-65536),
                                              jnp.float32)
                xr_v[b, r, pl.ds(s, 16)] = jnp.maximum(
                    xr_v[b, r, pl.ds(s, 16)] + lo, 0.0)
                xr_v[b, r, pl.ds(s + 16, 16)] = jnp.maximum(
                    xr_v[b, r, pl.ds(s + 16, 16)] + hi, 0.0)
            return c2
        lax.fori_loop(0, CH, row, 0)
        pltpu.sync_copy(xr_v.at[b], pooled_sh.at[idx_v.at[b, 1]], add=True)

        @pl.when(c + NB < NFULL)
        def _():
            start_idx(c + NB, b)

    start_idx(0, 0)
    start_idx(1, 1)
    start_idx(2, 2)
    wait_idx(0)
    start_data(0, 0)
    wait_idx(1)
    start_data(1, 1)

    def triple(j3, carry):
        process_chunk(3 * j3, 0)
        process_chunk(3 * j3 + 1, 1)
        process_chunk(3 * j3 + 2, 2)
        return carry
    lax.fori_loop(0, NFULL // 3, triple, 0)
    process_chunk(NFULL - 2, (NFULL - 2) % NB)
    process_chunk(NFULL - 1, (NFULL - 1) % NB)

    plsc.subcore_barrier()
    for k in range(WB // CH):
        r0 = sid * WB + k * CH

        @pl.when(r0 + CH <= N)
        def _():
            pltpu.sync_copy(pooled_sh.at[pl.ds(r0, CH)],
                            out_hbm.at[cid, pl.ds(r0, CH)])


def _sc_pool(idx3, he, x):
    mesh = plsc.VectorSubcoreMesh(core_axis_name="c", subcore_axis_name="s")
    f = pl.kernel(
        _sc_pool_body,
        out_type=jax.ShapeDtypeStruct((NC, N, D), jnp.float32),
        mesh=mesh,
        scratch_types=[
            pltpu.VMEM((NB, 2, CH), jnp.int32),
            pltpu.VMEM((NB, CH, D), jnp.float32),
            pltpu.VMEM((CH * D // 2,), jnp.int32),
            pltpu.VMEM((CH * D // 2,), jnp.int32),
            pltpu.VMEM((CH * D // 2,), jnp.int32),
            pltpu.VMEM_SHARED((N, D), jnp.float32),
        ] + [pltpu.SemaphoreType.DMA] * 9,
    )
    return f(idx3, he, x)


def _node_body(eps_ref, x_ref, p_ref, w1_ref, b1_ref, w2_ref, b2_ref, out_ref):
    z = eps_ref[0, 0] * x_ref[...] + p_ref[0] + p_ref[1]
    h = jnp.maximum(
        jnp.dot(z, w1_ref[...], preferred_element_type=jnp.float32)
        + b1_ref[...], 0.0)
    out_ref[...] = (
        jnp.dot(h, w2_ref[...], preferred_element_type=jnp.float32)
        + b2_ref[...]
    )


def _node_update(epsp, x, pooled2, w1, b1, w2, b2):
    return pl.pallas_call(
        _node_body,
        grid=(N // BNODE,),
        in_specs=[
            pl.BlockSpec(memory_space=pltpu.SMEM),
            pl.BlockSpec((BNODE, D), lambda i: (i, 0)),
            pl.BlockSpec((NC, BNODE, D), lambda i: (0, i, 0)),
            pl.BlockSpec((D, 2 * D), lambda i: (0, 0)),
            pl.BlockSpec((1, 2 * D), lambda i: (0, 0)),
            pl.BlockSpec((2 * D, D), lambda i: (0, 0)),
            pl.BlockSpec((1, D), lambda i: (0, 0)),
        ],
        out_specs=pl.BlockSpec((BNODE, D), lambda i: (i, 0)),
        out_shape=jax.ShapeDtypeStruct((N, D), jnp.float32),
    )(epsp, x, pooled2, w1, b1, w2, b2)


def kernel(x, edge_index, edge_attr, We1, be1, We2, be2, eps, Wm1, bm1,
           gamma1, beta1, mu1, var1, Wm2, bm2, gamma2, beta2, mu2, var2):
    w_e = (We1 @ We2)[:, _PERM]
    b_e = (be1 @ We2 + be2)[_PERM]
    scale1 = gamma1 / jnp.sqrt(var1 + BN_EPS)
    w1 = Wm1 * scale1[None, :]
    b1 = (bm1 - mu1) * scale1 + beta1
    scale2 = gamma2 / jnp.sqrt(var2 + BN_EPS)
    w2 = Wm2 * scale2[None, :]
    b2 = (bm2 - mu2) * scale2 + beta2
    epsp = (1.0 + eps).reshape(1, 1)

    idx3 = jnp.stack([edge_index[0].reshape(NW * NFULL, CH),
                      edge_index[1].reshape(NW * NFULL, CH)], axis=1)

    he = lax.bitcast_convert_type(
        _compute_he(edge_attr, w_e, b_e[None, :]).reshape(E * D // 2, 2),
        jnp.int32)
    pooled2 = _sc_pool(idx3, he, x)
    return _node_update(epsp, x, pooled2, w1, b1[None, :], w2, b2[None, :])
